# Initial kernel scaffold; baseline (speedup 1.0000x reference)
import jax
import jax.numpy as jnp
from jax.experimental import pallas as pl


def kernel(coords, covariances, colors, alphas, W, K):
    raise NotImplementedError("write your pallas kernel here")


# speedup vs baseline: 45.2008x; 45.2008x over previous
import functools

import jax
import jax.numpy as jnp
from jax import lax
from jax.experimental import pallas as pl
from jax.experimental.pallas import tpu as pltpu
from jax.experimental.pallas import tpu_sc as plsc

IMG_H, IMG_W = 512, 512
HW = IMG_H * IMG_W
NSIGMA = 2
FOOT = 2 * NSIGMA + 1
P = FOOT * FOOT
NPTS = 500000

NW = 32
BLK = 128
BLKS_PER_W = 123
PTS_PER_W = BLKS_PER_W * BLK
NPAD = NW * PTS_PER_W
TCB = 4096
NPARAM = 12


def _project_body(ct, cv, co, al, w_ref, k_ref, out_ref):
    x0 = ct[0, :]
    x1 = ct[1, :]
    x2 = ct[2, :]
    r00 = w_ref[0, 0]; r01 = w_ref[0, 1]; r02 = w_ref[0, 2]; t0 = w_ref[0, 3]
    r10 = w_ref[1, 0]; r11 = w_ref[1, 1]; r12 = w_ref[1, 2]; t1 = w_ref[1, 3]
    r20 = w_ref[2, 0]; r21 = w_ref[2, 1]; r22 = w_ref[2, 2]; t2 = w_ref[2, 3]
    fx = k_ref[0, 0]; cx = k_ref[0, 2]
    fy = k_ref[1, 1]; cy = k_ref[1, 2]

    x = r00 * x0 + r01 * x1 + r02 * x2 + t0
    y = r10 * x0 + r11 * x1 + r12 * x2 + t1
    z = r20 * x0 + r21 * x1 + r22 * x2 + t2
    z_safe = jnp.maximum(z, 1e-3)
    u = fx * x / z_safe + cx
    v = fy * y / z_safe + cy

    j00 = fx / z_safe
    j02 = -fx * x / (z_safe * z_safe)
    j11 = fy / z_safe
    j12 = -fy * y / (z_safe * z_safe)
    m00 = j00 * r00 + j02 * r20
    m01 = j00 * r01 + j02 * r21
    m02 = j00 * r02 + j02 * r22
    m10 = j11 * r10 + j12 * r20
    m11 = j11 * r11 + j12 * r21
    m12 = j11 * r12 + j12 * r22

    c00 = cv[0, :]; c01 = cv[1, :]; c02 = cv[2, :]
    c11 = cv[3, :]; c12 = cv[4, :]; c22 = cv[5, :]
    s0x = m00 * c00 + m01 * c01 + m02 * c02
    s0y = m00 * c01 + m01 * c11 + m02 * c12
    s0z = m00 * c02 + m01 * c12 + m02 * c22
    s1x = m10 * c00 + m11 * c01 + m12 * c02
    s1y = m10 * c01 + m11 * c11 + m12 * c12
    s1z = m10 * c02 + m11 * c12 + m12 * c22
    a = s0x * m00 + s0y * m01 + s0z * m02 + 1e-2
    b = s0x * m10 + s0y * m11 + s0z * m12
    c = s1x * m10 + s1y * m11 + s1z * m12 + 1e-2
    det = a * c - b * b
    inv_a = c / det
    inv_b = -b / det
    inv_c = a / det

    pu = jnp.round(u)
    pv = jnp.round(v)
    opac = jax.nn.sigmoid(al[0, :])
    opac = jnp.where(z > 0.1, opac, 0.0)

    out_ref[0, :] = u
    out_ref[1, :] = v
    out_ref[2, :] = pu
    out_ref[3, :] = pv
    out_ref[4, :] = inv_a
    out_ref[5, :] = inv_b
    out_ref[6, :] = inv_c
    out_ref[7, :] = opac
    out_ref[8, :] = jax.nn.sigmoid(co[0, :])
    out_ref[9, :] = jax.nn.sigmoid(co[1, :])
    out_ref[10, :] = jax.nn.sigmoid(co[2, :])
    out_ref[11, :] = jnp.zeros_like(u)


def _project(coords_t, covs_t, colors_t, alphas_t, W, K):
    grid = NPAD // TCB
    return pl.pallas_call(
        _project_body,
        grid=(grid,),
        in_specs=[
            pl.BlockSpec((3, TCB), lambda i: (0, i)),
            pl.BlockSpec((6, TCB), lambda i: (0, i)),
            pl.BlockSpec((3, TCB), lambda i: (0, i)),
            pl.BlockSpec((1, TCB), lambda i: (0, i)),
            pl.BlockSpec(memory_space=pltpu.SMEM),
            pl.BlockSpec(memory_space=pltpu.SMEM),
        ],
        out_specs=pl.BlockSpec((NPARAM, TCB), lambda i: (0, i)),
        out_shape=jax.ShapeDtypeStruct((NPARAM, NPAD), jnp.float32),
    )(coords_t, covs_t, colors_t, alphas_t, W, K)


_OFFS = [(float(dx), float(dy))
         for dy in range(-NSIGMA, NSIGMA + 1)
         for dx in range(-NSIGMA, NSIGMA + 1)]

HW16 = HW // 16


def _splat_body(params_hbm, zeros_hbm, out_hbm, pblk, svals, sidx,
                acc_r, acc_g, acc_b, acc_w):
    cid = lax.axis_index("c")
    sid = lax.axis_index("s")
    wid = cid * 16 + sid
    base = wid * PTS_PER_W

    zslc = pl.ds(sid * HW16, HW16)
    pltpu.sync_copy(zeros_hbm.at[zslc], acc_r.at[zslc])
    pltpu.sync_copy(zeros_hbm.at[zslc], acc_g.at[zslc])
    pltpu.sync_copy(zeros_hbm.at[zslc], acc_b.at[zslc])
    pltpu.sync_copy(zeros_hbm.at[zslc], acc_w.at[zslc])
    plsc.subcore_barrier()

    def block(bi, _):
        off = base + bi * BLK
        pltpu.sync_copy(params_hbm.at[:, pl.ds(off, BLK)], pblk)

        def chunk(ci, _):
            pts = ci * 16
            sl = pl.ds(pts, 16)
            u = pblk[0, sl]
            v = pblk[1, sl]
            pu = pblk[2, sl]
            pv = pblk[3, sl]
            ia = pblk[4, sl]
            ib2 = 2.0 * pblk[5, sl]
            ic = pblk[6, sl]
            opac = pblk[7, sl]
            colr = pblk[8, sl]
            colg = pblk[9, sl]
            colb = pblk[10, sl]
            for o, (dx, dy) in enumerate(_OFFS):
                px = pu + dx
                py = pv + dy
                du = px - u
                dv = py - v
                quad = ia * du * du + ib2 * du * dv + ic * dv * dv
                w = opac * jnp.exp(-0.5 * quad)
                inb = ((px >= 0.0) & (px <= float(IMG_W - 1))
                       & (py >= 0.0) & (py <= float(IMG_H - 1)))
                w = jnp.where(inb, w, 0.0)
                pxc = jnp.clip(px, 0.0, float(IMG_W - 1))
                pyc = jnp.clip(py, 0.0, float(IMG_H - 1))
                idxv = (pyc * float(IMG_W) + pxc).astype(jnp.int32)
                osl = pl.ds(o * BLK + pts, 16)
                svals[0, osl] = w * colr
                svals[1, osl] = w * colg
                svals[2, osl] = w * colb
                svals[3, osl] = w
                sidx[o, sl] = idxv
            return 0

        lax.fori_loop(0, BLK // 16, chunk, 0)
        for o in range(P):
            ssl = pl.ds(o * BLK, BLK)
            idx = sidx.at[o]
            pltpu.sync_copy(svals.at[0, ssl], acc_r.at[idx], add=True)
            pltpu.sync_copy(svals.at[1, ssl], acc_g.at[idx], add=True)
            pltpu.sync_copy(svals.at[2, ssl], acc_b.at[idx], add=True)
            pltpu.sync_copy(svals.at[3, ssl], acc_w.at[idx], add=True)
        return 0

    lax.fori_loop(0, BLKS_PER_W, block, 0)
    plsc.subcore_barrier()
    oslc = pl.ds(sid * HW16, HW16)
    pltpu.sync_copy(acc_r.at[oslc], out_hbm.at[cid, 0, oslc])
    pltpu.sync_copy(acc_g.at[oslc], out_hbm.at[cid, 1, oslc])
    pltpu.sync_copy(acc_b.at[oslc], out_hbm.at[cid, 2, oslc])
    pltpu.sync_copy(acc_w.at[oslc], out_hbm.at[cid, 3, oslc])


def _splat(params, zeros):
    mesh = plsc.VectorSubcoreMesh(core_axis_name="c", subcore_axis_name="s")
    f = functools.partial(
        pl.kernel,
        out_type=jax.ShapeDtypeStruct((2, 4, HW), jnp.float32),
        mesh=mesh,
        scratch_types=[
            pltpu.VMEM((NPARAM, BLK), jnp.float32),
            pltpu.VMEM((4, P * BLK), jnp.float32),
            pltpu.VMEM((P, BLK), jnp.int32),
            pltpu.VMEM_SHARED((HW,), jnp.float32),
            pltpu.VMEM_SHARED((HW,), jnp.float32),
            pltpu.VMEM_SHARED((HW,), jnp.float32),
            pltpu.VMEM_SHARED((HW,), jnp.float32),
        ],
    )(_splat_body)
    return f(params, zeros)


PIX_PER_W = HW // NW
PIXBLK = 512


def _combine_body(parts_hbm, out_hbm, buf, stage):
    cid = lax.axis_index("c")
    sid = lax.axis_index("s")
    wid = cid * 16 + sid

    def block(bi, _):
        pix0 = wid * PIX_PER_W + bi * PIXBLK
        psl = pl.ds(pix0, PIXBLK)
        for c in range(2):
            for ch in range(4):
                pltpu.sync_copy(parts_hbm.at[c, ch, psl], buf.at[c * 4 + ch])

        def group(gi, _):
            sl = pl.ds(gi * 16, 16)
            wsum = buf[3, sl] + buf[7, sl] + 1e-8
            stage[0, sl] = (buf[0, sl] + buf[4, sl]) / wsum
            stage[1, sl] = (buf[1, sl] + buf[5, sl]) / wsum
            stage[2, sl] = (buf[2, sl] + buf[6, sl]) / wsum
            return 0

        lax.fori_loop(0, PIXBLK // 16, group, 0)
        pltpu.sync_copy(stage, out_hbm.at[:, psl])
        return 0

    lax.fori_loop(0, PIX_PER_W // PIXBLK, block, 0)


def _combine(parts):
    mesh = plsc.VectorSubcoreMesh(core_axis_name="c", subcore_axis_name="s")
    f = functools.partial(
        pl.kernel,
        out_type=jax.ShapeDtypeStruct((3, HW), jnp.float32),
        mesh=mesh,
        scratch_types=[
            pltpu.VMEM((8, PIXBLK), jnp.float32),
            pltpu.VMEM((3, PIXBLK), jnp.float32),
        ],
    )(_combine_body)
    return f(parts)


def kernel(coords, covariances, colors, alphas, W, K):
    n = coords.shape[0]
    pad = NPAD - n
    coords_t = jnp.pad(coords, ((0, pad), (0, 0))).T
    cov6 = covariances.reshape(n, 9)[:, jnp.array([0, 1, 2, 4, 5, 8])]
    covs_t = jnp.pad(cov6, ((0, pad), (0, 0))).T
    colors_t = jnp.pad(colors, ((0, pad), (0, 0))).T
    alphas_t = jnp.pad(alphas, (0, pad), constant_values=-1e4)[None, :]

    params = _project(coords_t, covs_t, colors_t, alphas_t, W, K)
    zeros = jnp.zeros((HW,), jnp.float32)
    parts = _splat(params, zeros)
    img = _combine(parts)
    return img.T.reshape(IMG_H, IMG_W, 3)
